# baseline (speedup 1.0000x reference)
import math

import jax
import jax.numpy as jnp
from jax import lax
from jax.experimental import pallas as pl
from jax.experimental.pallas import tpu as pltpu

_TEMP = 10000.0
_SCALE = 2.0 * math.pi
_EPS_POS = 1e-6
_EPS_LN = 1e-12
_HALF_PI = 0.5 * math.pi


def _body(tv_ref, tq_ref, tt_col_ref, tt_row_ref, w_col_ref, b_col_ref,
          w_row_ref, b_row_ref, otv_ref, otq_ref, pos2d_ref, pos1d_ref):
    b = pl.program_id(0)
    i = pl.program_id(1)
    C, H, W = pos2d_ref.shape
    N = pos1d_ref.shape[0]
    Hh = tv_ref.shape[2]

    @pl.when(jnp.logical_and(b == 0, i == 0))
    def _init():
        ci = lax.broadcasted_iota(jnp.int32, (C, H, W), 0)
        hi = lax.broadcasted_iota(jnp.int32, (C, H, W), 1)
        wi = lax.broadcasted_iota(jnp.int32, (C, H, W), 2)
        half = C // 2
        is_y = ci < half
        embed = jnp.where(is_y,
                          (hi + 1).astype(jnp.float32) * (_SCALE / (H + _EPS_POS)),
                          (wi + 1).astype(jnp.float32) * (_SCALE / (W + _EPS_POS)))
        j = jnp.where(is_y, ci, ci - half)
        expo = (2.0 / half) * (j // 2).astype(jnp.float32)
        inv_dim_t = jnp.exp(expo * (-math.log(_TEMP)))
        phase = (ci % 2).astype(jnp.float32) * _HALF_PI
        pos = jnp.sin(embed * inv_dim_t + phase)
        tt1 = tt_col_ref[:, 1:2]
        pos2d_ref[...] = pos + tt1[:, :, None]

        ni = lax.broadcasted_iota(jnp.int32, (N, C), 0).astype(jnp.float32)
        cj = lax.broadcasted_iota(jnp.int32, (N, C), 1)
        expo1 = (2.0 / C) * (cj // 2).astype(jnp.float32)
        inv_dim_t1 = jnp.exp(expo1 * (-math.log(_TEMP)))
        phase1 = (cj % 2).astype(jnp.float32) * _HALF_PI
        pos1d_ref[...] = jnp.sin(ni * inv_dim_t1 + phase1) + tt_row_ref[0:1, :]

    t = tv_ref[0] + pos2d_ref[:, pl.ds(i * Hh, Hh), :]
    s = jnp.sum(t, axis=0, keepdims=True)
    sq = jnp.sum(t * t, axis=0, keepdims=True)
    mu = s * (1.0 / C)
    var = sq * (1.0 / C) - mu * mu
    inv = lax.rsqrt(var + _EPS_LN)
    w3 = w_col_ref[...]
    b3 = b_col_ref[...]
    otv_ref[0] = (t - mu) * inv * w3[:, :, None] + b3[:, :, None]

    @pl.when(i == 0)
    def _tq():
        q = tq_ref[0] + pos1d_ref[...]
        mu1 = jnp.mean(q, axis=1, keepdims=True)
        qc = q - mu1
        var1 = jnp.mean(qc * qc, axis=1, keepdims=True)
        otq_ref[0] = (qc * lax.rsqrt(var1 + _EPS_LN) * w_row_ref[...]
                      + b_row_ref[...])


def kernel(input_tv, input_tq, tv_positions, tq_positions, token_type_table,
           ln_weight, ln_bias):
    B, C, H, W = input_tv.shape
    N = input_tq.shape[1]
    Hh = H // 2

    tt_col = token_type_table.T
    w_col = ln_weight.reshape(C, 1)
    b_col = ln_bias.reshape(C, 1)
    w_row = ln_weight.reshape(1, C)
    b_row = ln_bias.reshape(1, C)

    otv, otq = pl.pallas_call(
        _body,
        grid=(B, 2),
        in_specs=[
            pl.BlockSpec((1, C, Hh, W), lambda b, i: (b, 0, i, 0)),
            pl.BlockSpec((1, N, C), lambda b, i: (b, 0, 0)),
            pl.BlockSpec((C, 2), lambda b, i: (0, 0)),
            pl.BlockSpec((2, C), lambda b, i: (0, 0)),
            pl.BlockSpec((C, 1), lambda b, i: (0, 0)),
            pl.BlockSpec((C, 1), lambda b, i: (0, 0)),
            pl.BlockSpec((1, C), lambda b, i: (0, 0)),
            pl.BlockSpec((1, C), lambda b, i: (0, 0)),
        ],
        out_specs=[
            pl.BlockSpec((1, C, Hh, W), lambda b, i: (b, 0, i, 0)),
            pl.BlockSpec((1, N, C), lambda b, i: (b, 0, 0)),
        ],
        out_shape=[
            jax.ShapeDtypeStruct((B, C, H, W), jnp.float32),
            jax.ShapeDtypeStruct((B, N, C), jnp.float32),
        ],
        scratch_shapes=[
            pltpu.VMEM((C, H, W), jnp.float32),
            pltpu.VMEM((N, C), jnp.float32),
        ],
        compiler_params=pltpu.CompilerParams(
            dimension_semantics=("arbitrary", "arbitrary"),
        ),
    )(input_tv, input_tq, tt_col, token_type_table, w_col, b_col, w_row,
      b_row)

    return otv, otq


# speedup vs baseline: 1.0734x; 1.0734x over previous
import math

import jax
import jax.numpy as jnp
from jax import lax
from jax.experimental import pallas as pl
from jax.experimental.pallas import tpu as pltpu

_TEMP = 10000.0
_SCALE = 2.0 * math.pi
_EPS_POS = 1e-6
_EPS_LN = 1e-12
_HALF_PI = 0.5 * math.pi
_LANES = 128


def _body(tv_ref, tq_ref, tt_col_ref, tt_row_ref, w_col_ref, b_col_ref,
          w_row_ref, b_row_ref, otv_ref, otq_ref,
          pos2d_ref, pos1d_ref, w_exp_ref, b_exp_ref):
    b = pl.program_id(0)
    C, S, L = pos2d_ref.shape
    N = pos1d_ref.shape[0]
    HW = S * L
    H = 32
    W = HW // H

    @pl.when(b == 0)
    def _init():
        ci = lax.broadcasted_iota(jnp.int32, (C, S, L), 0)
        si = lax.broadcasted_iota(jnp.int32, (C, S, L), 1)
        li = lax.broadcasted_iota(jnp.int32, (C, S, L), 2)
        hw = si * L + li
        half = C // 2
        is_y = ci < half
        embed = jnp.where(is_y,
                          (hw // W + 1).astype(jnp.float32) * (_SCALE / (H + _EPS_POS)),
                          (hw % W + 1).astype(jnp.float32) * (_SCALE / (W + _EPS_POS)))
        j = jnp.where(is_y, ci, ci - half)
        expo = (2.0 / half) * (j // 2).astype(jnp.float32)
        inv_dim_t = jnp.exp(expo * (-math.log(_TEMP)))
        phase = (ci % 2).astype(jnp.float32) * _HALF_PI
        pos = jnp.sin(embed * inv_dim_t + phase)
        tt1 = tt_col_ref[:, 1:2]
        pos2d_ref[...] = pos + tt1[:, :, None]

        w1 = w_col_ref[...]
        b1 = b_col_ref[...]
        w_exp_ref[...] = jnp.broadcast_to(w1[:, :, None], (C, S, L))
        b_exp_ref[...] = jnp.broadcast_to(b1[:, :, None], (C, S, L))

        ni = lax.broadcasted_iota(jnp.int32, (N, C), 0).astype(jnp.float32)
        cj = lax.broadcasted_iota(jnp.int32, (N, C), 1)
        expo1 = (2.0 / C) * (cj // 2).astype(jnp.float32)
        inv_dim_t1 = jnp.exp(expo1 * (-math.log(_TEMP)))
        phase1 = (cj % 2).astype(jnp.float32) * _HALF_PI
        pos1d_ref[...] = jnp.sin(ni * inv_dim_t1 + phase1) + tt_row_ref[0:1, :]

    t = jnp.reshape(tv_ref[...], (C, S, L)) + pos2d_ref[...]
    s = jnp.sum(t, axis=0, keepdims=True)
    sq = jnp.sum(t * t, axis=0, keepdims=True)
    mu = s * (1.0 / C)
    var = sq * (1.0 / C) - mu * mu
    inv = lax.rsqrt(var + _EPS_LN)
    y = (t - mu) * inv * w_exp_ref[...] + b_exp_ref[...]
    otv_ref[...] = jnp.reshape(y, (C * S, L))

    q = tq_ref[...] + pos1d_ref[...]
    mu1 = jnp.mean(q, axis=1, keepdims=True)
    qc = q - mu1
    var1 = jnp.mean(qc * qc, axis=1, keepdims=True)
    otq_ref[...] = (qc * lax.rsqrt(var1 + _EPS_LN) * w_row_ref[...]
                    + b_row_ref[...])


def kernel(input_tv, input_tq, tv_positions, tq_positions, token_type_table,
           ln_weight, ln_bias):
    B, C, H, W = input_tv.shape
    N = input_tq.shape[1]
    HW = H * W
    S = HW // _LANES
    ROWS = C * S

    tvL = input_tv.reshape(B * ROWS, _LANES)
    tq2 = input_tq.reshape(B * N, C)
    tt_col = token_type_table.T
    w_col = ln_weight.reshape(C, 1)
    b_col = ln_bias.reshape(C, 1)
    w_row = ln_weight.reshape(1, C)
    b_row = ln_bias.reshape(1, C)

    otv, otq = pl.pallas_call(
        _body,
        grid=(B,),
        in_specs=[
            pl.BlockSpec((ROWS, _LANES), lambda b: (b, 0)),
            pl.BlockSpec((N, C), lambda b: (b, 0)),
            pl.BlockSpec((C, 2), lambda b: (0, 0)),
            pl.BlockSpec((2, C), lambda b: (0, 0)),
            pl.BlockSpec((C, 1), lambda b: (0, 0)),
            pl.BlockSpec((C, 1), lambda b: (0, 0)),
            pl.BlockSpec((1, C), lambda b: (0, 0)),
            pl.BlockSpec((1, C), lambda b: (0, 0)),
        ],
        out_specs=[
            pl.BlockSpec((ROWS, _LANES), lambda b: (b, 0)),
            pl.BlockSpec((N, C), lambda b: (b, 0)),
        ],
        out_shape=[
            jax.ShapeDtypeStruct((B * ROWS, _LANES), jnp.float32),
            jax.ShapeDtypeStruct((B * N, C), jnp.float32),
        ],
        scratch_shapes=[
            pltpu.VMEM((C, S, _LANES), jnp.float32),
            pltpu.VMEM((N, C), jnp.float32),
            pltpu.VMEM((C, S, _LANES), jnp.float32),
            pltpu.VMEM((C, S, _LANES), jnp.float32),
        ],
        compiler_params=pltpu.CompilerParams(
            dimension_semantics=("arbitrary",),
        ),
    )(tvL, tq2, tt_col, token_type_table, w_col, b_col, w_row, b_row)

    return otv.reshape(B, C, H, W), otq.reshape(B, N, C)


# speedup vs baseline: 9.1637x; 8.5372x over previous
import math

import jax
import jax.numpy as jnp
from jax import lax
from jax.experimental import pallas as pl
from jax.experimental.pallas import tpu as pltpu

_TEMP = 10000.0
_SCALE = 2.0 * math.pi
_EPS_POS = 1e-6
_EPS_LN = 1e-12
_HALF_PI = 0.5 * math.pi


def _row_layernorm(x, w_row, b_row):
    mu = jnp.mean(x, axis=1, keepdims=True)
    xc = x - mu
    var = jnp.mean(xc * xc, axis=1, keepdims=True)
    return xc * lax.rsqrt(var + _EPS_LN) * w_row + b_row


def _body(tv_ref, tq_ref, tt_row_ref, w_row_ref, b_row_ref,
          otv_ref, otq_ref, pos2d_ref, pos1d_ref):
    b = pl.program_id(0)
    HW, C = pos2d_ref.shape
    N = pos1d_ref.shape[0]
    H = 32
    W = HW // H

    @pl.when(b == 0)
    def _init():
        hwi = lax.broadcasted_iota(jnp.int32, (HW, C), 0)
        ci = lax.broadcasted_iota(jnp.int32, (HW, C), 1)
        half = C // 2
        is_y = ci < half
        embed = jnp.where(is_y,
                          (hwi // W + 1).astype(jnp.float32) * (_SCALE / (H + _EPS_POS)),
                          (hwi % W + 1).astype(jnp.float32) * (_SCALE / (W + _EPS_POS)))
        j = jnp.where(is_y, ci, ci - half)
        expo = (2.0 / half) * (j // 2).astype(jnp.float32)
        inv_dim_t = jnp.exp(expo * (-math.log(_TEMP)))
        phase = (ci % 2).astype(jnp.float32) * _HALF_PI
        pos2d_ref[...] = (jnp.sin(embed * inv_dim_t + phase)
                          + tt_row_ref[1:2, :])

        ni = lax.broadcasted_iota(jnp.int32, (N, C), 0).astype(jnp.float32)
        cj = lax.broadcasted_iota(jnp.int32, (N, C), 1)
        expo1 = (2.0 / C) * (cj // 2).astype(jnp.float32)
        inv_dim_t1 = jnp.exp(expo1 * (-math.log(_TEMP)))
        phase1 = (cj % 2).astype(jnp.float32) * _HALF_PI
        pos1d_ref[...] = jnp.sin(ni * inv_dim_t1 + phase1) + tt_row_ref[0:1, :]

    w = w_row_ref[...]
    bb = b_row_ref[...]
    otv_ref[...] = _row_layernorm(tv_ref[...] + pos2d_ref[...], w, bb)
    otq_ref[...] = _row_layernorm(tq_ref[...] + pos1d_ref[...], w, bb)


def kernel(input_tv, input_tq, tv_positions, tq_positions, token_type_table,
           ln_weight, ln_bias):
    B, C, H, W = input_tv.shape
    N = input_tq.shape[1]
    HW = H * W

    tvT = jnp.transpose(input_tv, (0, 2, 3, 1)).reshape(B * HW, C)
    tq2 = input_tq.reshape(B * N, C)
    w_row = ln_weight.reshape(1, C)
    b_row = ln_bias.reshape(1, C)

    otv, otq = pl.pallas_call(
        _body,
        grid=(B,),
        in_specs=[
            pl.BlockSpec((HW, C), lambda b: (b, 0)),
            pl.BlockSpec((N, C), lambda b: (b, 0)),
            pl.BlockSpec((2, C), lambda b: (0, 0)),
            pl.BlockSpec((1, C), lambda b: (0, 0)),
            pl.BlockSpec((1, C), lambda b: (0, 0)),
        ],
        out_specs=[
            pl.BlockSpec((HW, C), lambda b: (b, 0)),
            pl.BlockSpec((N, C), lambda b: (b, 0)),
        ],
        out_shape=[
            jax.ShapeDtypeStruct((B * HW, C), jnp.float32),
            jax.ShapeDtypeStruct((B * N, C), jnp.float32),
        ],
        scratch_shapes=[
            pltpu.VMEM((HW, C), jnp.float32),
            pltpu.VMEM((N, C), jnp.float32),
        ],
        compiler_params=pltpu.CompilerParams(
            dimension_semantics=("arbitrary",),
        ),
    )(tvT, tq2, token_type_table, w_row, b_row)

    otv4 = jnp.transpose(otv.reshape(B, H, W, C), (0, 3, 1, 2))
    return otv4, otq.reshape(B, N, C)


# speedup vs baseline: 11.2769x; 1.2306x over previous
import math

import jax
import jax.numpy as jnp
from jax import lax
from jax.experimental import pallas as pl
from jax.experimental.pallas import tpu as pltpu

_TEMP = 10000.0
_SCALE = 2.0 * math.pi
_EPS_POS = 1e-6
_EPS_LN = 1e-12
_HALF_PI = 0.5 * math.pi


def _row_layernorm(x, w_row, b_row):
    mu = jnp.mean(x, axis=1, keepdims=True)
    xc = x - mu
    var = jnp.mean(xc * xc, axis=1, keepdims=True)
    return xc * lax.rsqrt(var + _EPS_LN) * w_row + b_row


def _body(tv_ref, tq_ref, tt_row_ref, w_row_ref, b_row_ref,
          otv_ref, otq_ref, pos2d_ref, pos1d_ref):
    b = pl.program_id(0)
    HW, C = pos2d_ref.shape
    N = pos1d_ref.shape[0]
    H = 32
    W = HW // H

    @pl.when(b == 0)
    def _init():
        half = C // 2
        yi = lax.broadcasted_iota(jnp.int32, (H, half), 0)
        cj = lax.broadcasted_iota(jnp.int32, (H, half), 1)
        expo = (2.0 / half) * (cj // 2).astype(jnp.float32)
        inv_dim_t = jnp.exp(expo * (-math.log(_TEMP)))
        phase = (cj % 2).astype(jnp.float32) * _HALF_PI
        pos_y = (jnp.sin((yi + 1).astype(jnp.float32) * (_SCALE / (H + _EPS_POS))
                         * inv_dim_t + phase)
                 + tt_row_ref[1:2, 0:half])
        pos_x = (jnp.sin((yi + 1).astype(jnp.float32) * (_SCALE / (W + _EPS_POS))
                         * inv_dim_t + phase)
                 + tt_row_ref[1:2, half:C])
        pos2d_ref[:, 0:half] = jnp.repeat(pos_y, W, axis=0)
        pos2d_ref[:, half:C] = jnp.tile(pos_x, (H, 1))

        ni = lax.broadcasted_iota(jnp.int32, (N, C), 0).astype(jnp.float32)
        cj = lax.broadcasted_iota(jnp.int32, (N, C), 1)
        expo1 = (2.0 / C) * (cj // 2).astype(jnp.float32)
        inv_dim_t1 = jnp.exp(expo1 * (-math.log(_TEMP)))
        phase1 = (cj % 2).astype(jnp.float32) * _HALF_PI
        pos1d_ref[...] = jnp.sin(ni * inv_dim_t1 + phase1) + tt_row_ref[0:1, :]

    w = w_row_ref[...]
    bb = b_row_ref[...]
    otv_ref[...] = _row_layernorm(tv_ref[...] + pos2d_ref[...], w, bb)
    otq_ref[...] = _row_layernorm(tq_ref[...] + pos1d_ref[...], w, bb)


def kernel(input_tv, input_tq, tv_positions, tq_positions, token_type_table,
           ln_weight, ln_bias):
    B, C, H, W = input_tv.shape
    N = input_tq.shape[1]
    HW = H * W

    tvT = jnp.transpose(input_tv, (0, 2, 3, 1)).reshape(B * HW, C)
    tq2 = input_tq.reshape(B * N, C)
    w_row = ln_weight.reshape(1, C)
    b_row = ln_bias.reshape(1, C)

    otv, otq = pl.pallas_call(
        _body,
        grid=(B,),
        in_specs=[
            pl.BlockSpec((HW, C), lambda b: (b, 0)),
            pl.BlockSpec((N, C), lambda b: (b, 0)),
            pl.BlockSpec((2, C), lambda b: (0, 0)),
            pl.BlockSpec((1, C), lambda b: (0, 0)),
            pl.BlockSpec((1, C), lambda b: (0, 0)),
        ],
        out_specs=[
            pl.BlockSpec((HW, C), lambda b: (b, 0)),
            pl.BlockSpec((N, C), lambda b: (b, 0)),
        ],
        out_shape=[
            jax.ShapeDtypeStruct((B * HW, C), jnp.float32),
            jax.ShapeDtypeStruct((B * N, C), jnp.float32),
        ],
        scratch_shapes=[
            pltpu.VMEM((HW, C), jnp.float32),
            pltpu.VMEM((N, C), jnp.float32),
        ],
        compiler_params=pltpu.CompilerParams(
            dimension_semantics=("arbitrary",),
        ),
    )(tvT, tq2, token_type_table, w_row, b_row)

    otv4 = jnp.transpose(otv.reshape(B, H, W, C), (0, 3, 1, 2))
    return otv4, otq.reshape(B, N, C)


# speedup vs baseline: 11.7143x; 1.0388x over previous
import math

import jax
import jax.numpy as jnp
from jax import lax
from jax.experimental import pallas as pl
from jax.experimental.pallas import tpu as pltpu

_TEMP = 10000.0
_SCALE = 2.0 * math.pi
_EPS_POS = 1e-6
_EPS_LN = 1e-12
_HALF_PI = 0.5 * math.pi


def _row_layernorm(x, w_row, b_row):
    mu = jnp.mean(x, axis=1, keepdims=True)
    xc = x - mu
    var = jnp.mean(xc * xc, axis=1, keepdims=True)
    return xc * lax.rsqrt(var + _EPS_LN) * w_row + b_row


def _body(tv_ref, tq_ref, params_ref, otv_ref, otq_ref, pos2d_ref, pos1d_ref):
    b = pl.program_id(0)
    HW, C = pos2d_ref.shape
    N = pos1d_ref.shape[0]
    H = 32
    W = HW // H

    @pl.when(b == 0)
    def _init():
        half = C // 2
        yi = lax.broadcasted_iota(jnp.int32, (H, half), 0)
        cj = lax.broadcasted_iota(jnp.int32, (H, half), 1)
        expo = (2.0 / half) * (cj // 2).astype(jnp.float32)
        inv_dim_t = jnp.exp(expo * (-math.log(_TEMP)))
        phase = (cj % 2).astype(jnp.float32) * _HALF_PI
        pos_y = (jnp.sin((yi + 1).astype(jnp.float32) * (_SCALE / (H + _EPS_POS))
                         * inv_dim_t + phase)
                 + params_ref[1:2, 0:half])
        pos_x = (jnp.sin((yi + 1).astype(jnp.float32) * (_SCALE / (W + _EPS_POS))
                         * inv_dim_t + phase)
                 + params_ref[1:2, half:C])
        pos2d_ref[:, 0:half] = jnp.repeat(pos_y, W, axis=0)
        pos2d_ref[:, half:C] = jnp.tile(pos_x, (H, 1))

        ni = lax.broadcasted_iota(jnp.int32, (N, C), 0).astype(jnp.float32)
        ck = lax.broadcasted_iota(jnp.int32, (N, C), 1)
        expo1 = (2.0 / C) * (ck // 2).astype(jnp.float32)
        inv_dim_t1 = jnp.exp(expo1 * (-math.log(_TEMP)))
        phase1 = (ck % 2).astype(jnp.float32) * _HALF_PI
        pos1d_ref[...] = jnp.sin(ni * inv_dim_t1 + phase1) + params_ref[0:1, :]

    w = params_ref[2:3, :]
    bb = params_ref[3:4, :]
    PB = tv_ref.shape[0] // HW
    tv3 = jnp.reshape(tv_ref[...], (PB, HW, C)) + pos2d_ref[...][None]
    mu = jnp.mean(tv3, axis=2, keepdims=True)
    xc = tv3 - mu
    var = jnp.mean(xc * xc, axis=2, keepdims=True)
    y3 = xc * lax.rsqrt(var + _EPS_LN) * w[None] + bb[None]
    otv_ref[...] = jnp.reshape(y3, (PB * HW, C))

    tq3 = jnp.reshape(tq_ref[...], (PB, N, C)) + pos1d_ref[...][None]
    mu1 = jnp.mean(tq3, axis=2, keepdims=True)
    qc = tq3 - mu1
    var1 = jnp.mean(qc * qc, axis=2, keepdims=True)
    q3 = qc * lax.rsqrt(var1 + _EPS_LN) * w[None] + bb[None]
    otq_ref[...] = jnp.reshape(q3, (PB * N, C))


def kernel(input_tv, input_tq, tv_positions, tq_positions, token_type_table,
           ln_weight, ln_bias):
    B, C, H, W = input_tv.shape
    N = input_tq.shape[1]
    HW = H * W
    PB = 2

    tvT = jnp.transpose(input_tv, (0, 2, 3, 1)).reshape(B * HW, C)
    tq2 = input_tq.reshape(B * N, C)
    params = jnp.concatenate(
        [token_type_table, ln_weight.reshape(1, C), ln_bias.reshape(1, C)],
        axis=0)

    otv, otq = pl.pallas_call(
        _body,
        grid=(B // PB,),
        in_specs=[
            pl.BlockSpec((PB * HW, C), lambda b: (b, 0)),
            pl.BlockSpec((PB * N, C), lambda b: (b, 0)),
            pl.BlockSpec((4, C), lambda b: (0, 0)),
        ],
        out_specs=[
            pl.BlockSpec((PB * HW, C), lambda b: (b, 0)),
            pl.BlockSpec((PB * N, C), lambda b: (b, 0)),
        ],
        out_shape=[
            jax.ShapeDtypeStruct((B * HW, C), jnp.float32),
            jax.ShapeDtypeStruct((B * N, C), jnp.float32),
        ],
        scratch_shapes=[
            pltpu.VMEM((HW, C), jnp.float32),
            pltpu.VMEM((N, C), jnp.float32),
        ],
        compiler_params=pltpu.CompilerParams(
            dimension_semantics=("arbitrary",),
        ),
    )(tvT, tq2, params)

    otv4 = jnp.transpose(otv.reshape(B, H, W, C), (0, 3, 1, 2))
    return otv4, otq.reshape(B, N, C)
